# Initial kernel scaffold; baseline (speedup 1.0000x reference)
import jax
import jax.numpy as jnp
from jax.experimental import pallas as pl


def kernel(inputs, edge_index, edge_vals):
    raise NotImplementedError("write your pallas kernel here")


# speedup vs baseline: 3.3440x; 3.3440x over previous
import functools

import jax
import jax.numpy as jnp
from jax import lax
from jax.experimental import pallas as pl
from jax.experimental.pallas import tpu as pltpu
from jax.experimental.pallas import tpu_sc as plsc

N = 10000
F = 128
T = 4
NTILES = 16
CHUNK = 128
N_PAD = 10240


def _sc_body(nchunks, xflat, cols_h, rows_h, vals_h, out_h,
             cols_v, rows_v, vals_v, gbuf, acc, sem):
    c = lax.axis_index("c")
    s = lax.axis_index("s")
    stripe = N_PAD // NTILES

    pltpu.sync_copy(cols_h.at[s], cols_v)
    pltpu.sync_copy(rows_h.at[s], rows_v)
    pltpu.sync_copy(vals_h.at[s], vals_v)

    for phase in range(T // 2):
        t = phase * 2 + c

        delta = c * N if phase == 0 else 2 * N

        def _cj(j, _):
            for k in range(CHUNK // 16):
                cols_v[j, pl.ds(16 * k, 16)] = (
                    cols_v[j, pl.ds(16 * k, 16)] + delta)
            return 0
        lax.fori_loop(0, nchunks, _cj, 0)

        def _zr(r, _):
            for k in range(F // 16):
                gbuf[r, pl.ds(16 * k, 16)] = jnp.zeros((16,), jnp.float32)
            return 0
        lax.fori_loop(0, CHUNK, _zr, 0)
        for z in range(stripe // CHUNK):
            pltpu.sync_copy(gbuf, acc.at[pl.ds(s * stripe + z * CHUNK, CHUNK)])

        plsc.subcore_barrier()

        def _chunk(j, _):
            pltpu.async_copy(xflat.at[cols_v.at[j]], gbuf, sem).wait()

            def _egroup(g, _):
                vv = vals_v[pl.ds(j * CHUNK + g * 16, 16)]
                for l in range(16):
                    v = vv[l]
                    i = g * 16 + l
                    for k in range(F // 16):
                        gbuf[i, pl.ds(16 * k, 16)] = (
                            gbuf[i, pl.ds(16 * k, 16)] * v)
                return 0
            lax.fori_loop(0, CHUNK // 16, _egroup, 0)

            pltpu.sync_copy(gbuf, acc.at[rows_v.at[j]], add=True)
            return 0
        lax.fori_loop(0, nchunks, _chunk, 0)

        plsc.subcore_barrier()

        last = N - (NTILES - 1) * stripe

        @pl.when(s < NTILES - 1)
        def _():
            pltpu.sync_copy(acc.at[pl.ds(s * stripe, stripe)],
                            out_h.at[t, pl.ds(s * stripe, stripe)])

        @pl.when(s == NTILES - 1)
        def _():
            pltpu.sync_copy(acc.at[pl.ds((NTILES - 1) * stripe, last)],
                            out_h.at[t, pl.ds((NTILES - 1) * stripe, last)])


@jax.jit
def _spmm_sc(xflat, cols3, rows3, vals3):
    nchunks = cols3.shape[1]
    kfn = functools.partial(
        pl.kernel,
        mesh=plsc.VectorSubcoreMesh(core_axis_name="c", subcore_axis_name="s"),
        out_type=jax.ShapeDtypeStruct((T, N, F), jnp.float32),
        scratch_types=[
            pltpu.VMEM((nchunks, CHUNK), jnp.int32),
            pltpu.VMEM((nchunks, CHUNK), jnp.int32),
            pltpu.VMEM((nchunks * CHUNK,), jnp.float32),
            pltpu.VMEM((CHUNK, F), jnp.float32),
            pltpu.VMEM_SHARED((N_PAD, F), jnp.float32),
            pltpu.SemaphoreType.DMA,
        ],
    )(functools.partial(_sc_body, nchunks))
    return kfn(xflat, cols3, rows3, vals3)


def kernel(inputs, edge_index, edge_vals):
    B = inputs.shape[0]
    E = edge_vals.shape[0]
    xflat = jnp.reshape(inputs, (B * T * N, F))

    per_tile = -(-E // NTILES)
    nchunks = -(-per_tile // CHUNK)
    ep = NTILES * nchunks * CHUNK
    pad = ep - E
    rows = jnp.pad(edge_index[0], (0, pad))
    cols = jnp.pad(edge_index[1], (0, pad))
    vals = jnp.pad(edge_vals, (0, pad))

    cols3 = jnp.reshape(cols, (NTILES, nchunks, CHUNK))
    rows3 = jnp.reshape(rows, (NTILES, nchunks, CHUNK))
    vals3 = jnp.reshape(vals, (NTILES, nchunks * CHUNK))

    out = _spmm_sc(xflat, cols3, rows3, vals3)
    return out[None]
